# Initial kernel scaffold; baseline (speedup 1.0000x reference)
import jax
import jax.numpy as jnp
from jax.experimental import pallas as pl


def kernel(x, seq_lens, pos_embedding):
    raise NotImplementedError("write your pallas kernel here")


# speedup vs baseline: 9.5439x; 9.5439x over previous
import jax
import jax.numpy as jnp
from jax.experimental import pallas as pl
from jax.experimental.pallas import tpu as pltpu

D_MODEL = 128
SEQ_LEN = 200
BATCH = 1024

B_BLK = 16


def _add_pe_body(x_ref, pe_ref, o_ref):
    o_ref[...] = x_ref[...] + pe_ref[...]


def kernel(x, seq_lens, pos_embedding):
    del seq_lens
    batch, seq_len, d = x.shape
    pe = pos_embedding[:seq_len][None, :, :]

    grid = (batch // B_BLK,)
    out = pl.pallas_call(
        _add_pe_body,
        grid=grid,
        in_specs=[
            pl.BlockSpec((B_BLK, seq_len, d), lambda i: (i, 0, 0)),
            pl.BlockSpec((1, seq_len, d), lambda i: (0, 0, 0)),
        ],
        out_specs=pl.BlockSpec((B_BLK, seq_len, d), lambda i: (i, 0, 0)),
        out_shape=jax.ShapeDtypeStruct((batch, seq_len, d), x.dtype),
        compiler_params=pltpu.CompilerParams(
            dimension_semantics=("arbitrary",),
        ),
    )(x, pe)
    return out
